# Initial kernel scaffold; baseline (speedup 1.0000x reference)
import jax
import jax.numpy as jnp
from jax.experimental import pallas as pl


def kernel(x, forward_shuffle_idx):
    raise NotImplementedError("write your pallas kernel here")


# speedup vs baseline: 2.8508x; 2.8508x over previous
import functools

import jax
import jax.numpy as jnp
from jax import lax
from jax.experimental import pallas as pl
from jax.experimental.pallas import tpu as pltpu
from jax.experimental.pallas import tpu_sc as plsc

B, S, D = 16, 4096, 256
NW = 32
ROWS_PER_W = B * S // NW
CHUNK = 128
NCHUNK = ROWS_PER_W // CHUNK
L = 16

_mesh = plsc.VectorSubcoreMesh(core_axis_name="c", subcore_axis_name="s")


@functools.partial(
    pl.kernel,
    mesh=_mesh,
    out_type=jax.ShapeDtypeStruct((B * S, D), jnp.float32),
    scratch_types=[
        pltpu.VMEM((NCHUNK, CHUNK), jnp.int32),
        pltpu.VMEM((CHUNK, D), jnp.float32),
        pltpu.VMEM((CHUNK, D), jnp.float32),
        pltpu.SemaphoreType.DMA,
        pltpu.SemaphoreType.DMA,
    ],
)
def _zcurve_gather(x_hbm, idx_hbm, out_hbm, gidx_v, rows0_v, rows1_v, sem0, sem1):
    wid = lax.axis_index("s") * 2 + lax.axis_index("c")
    b = wid // 2
    h = wid % 2
    out_base = wid * ROWS_PER_W

    pltpu.sync_copy(idx_hbm.at[pl.ds(h * NCHUNK, NCHUNK)], gidx_v)
    off = b * S
    for c in range(NCHUNK):
        for s in range(CHUNK // L):
            gidx_v[c, pl.ds(s * L, L)] = gidx_v[c, pl.ds(s * L, L)] + off

    bufs = (rows0_v, rows1_v)
    sems = (sem0, sem1)
    copies = [None, None]
    for c in range(NCHUNK):
        p = c % 2
        copies[p] = pltpu.async_copy(x_hbm.at[gidx_v.at[c]], bufs[p], sems[p])
        if c > 0:
            copies[1 - p].wait()
            pltpu.sync_copy(
                bufs[1 - p], out_hbm.at[pl.ds(out_base + (c - 1) * CHUNK, CHUNK)]
            )
        if c == NCHUNK - 1:
            copies[p].wait()
            pltpu.sync_copy(
                bufs[p], out_hbm.at[pl.ds(out_base + c * CHUNK, CHUNK)]
            )


def kernel(x, forward_shuffle_idx):
    x2 = x.reshape(B * S, D)
    idx2d = forward_shuffle_idx.reshape(NW, CHUNK)
    out = _zcurve_gather(x2, idx2d)
    return out.reshape(B, S, D)
